# baseline (speedup 1.0000x reference)
import functools

import jax
import jax.numpy as jnp
from jax import lax
from jax.experimental import pallas as pl
from jax.experimental.pallas import tpu as pltpu
from jax.experimental.pallas import tpu_sc as plsc

VOCAB = 100000
MAX_LEN = 200
EMBED_DIM = 64
BATCH = 1024
SEQ = 200

NUM_CORES = 2
NUM_SUBCORES = 16
NUM_WORKERS = NUM_CORES * NUM_SUBCORES
ROWS_PER_WORKER = BATCH * SEQ // NUM_WORKERS
CHUNK = 40
CHUNKS_PER_WORKER = ROWS_PER_WORKER // CHUNK


def _sc_kernel_body(x_hbm, tok_hbm, pos_hbm, out_hbm,
                    idx_v, buf0, buf1, gsem0, gsem1, psem0, psem1):
    wid = lax.axis_index("s") * NUM_CORES + lax.axis_index("c")
    base = wid * ROWS_PER_WORKER

    pltpu.sync_copy(x_hbm.at[pl.ds(base, ROWS_PER_WORKER)], idx_v)

    bufs = (buf0, buf1)
    gsems = (gsem0, gsem1)
    psems = (psem0, psem1)

    def start_prefill(c, slot):
        s0 = lax.rem(c, SEQ // CHUNK) * CHUNK
        pltpu.async_copy(pos_hbm.at[pl.ds(s0, CHUNK)], bufs[slot],
                         psems[slot])

    def wait_prefill(slot):
        pltpu.make_async_copy(pos_hbm.at[pl.ds(0, CHUNK)], bufs[slot],
                              psems[slot]).wait()

    def start_gather_add(c, slot):
        idx = idx_v.at[pl.ds(c * CHUNK, CHUNK)]
        pltpu.async_copy(tok_hbm.at[idx], bufs[slot], gsems[slot], add=True)

    def wait_gather(slot):
        pltpu.make_async_copy(tok_hbm.at[idx_v.at[pl.ds(0, CHUNK)]],
                              bufs[slot], gsems[slot]).wait()

    def emit(c, slot):
        pltpu.sync_copy(bufs[slot], out_hbm.at[pl.ds(base + c * CHUNK, CHUNK)])

    start_prefill(0, 0)
    wait_prefill(0)
    start_gather_add(0, 0)
    start_prefill(1, 1)

    def step(c, carry):
        for b in range(2):
            slot = b
            other = 1 - b

            @pl.when(c + b + 1 < CHUNKS_PER_WORKER)
            def _next_gather():
                wait_prefill(other)
                start_gather_add(c + b + 1, other)

            wait_gather(slot)
            emit(c + b, slot)

            @pl.when(c + b + 2 < CHUNKS_PER_WORKER)
            def _next_prefill():
                start_prefill(c + b + 2, slot)
        return carry

    def outer(g, carry):
        return step(g * 2, carry)

    lax.fori_loop(0, CHUNKS_PER_WORKER // 2, outer, 0)


@jax.jit
def kernel(x, token_table, pos_table):
    x_flat = x.reshape(BATCH * SEQ).astype(jnp.int32)

    mesh = plsc.VectorSubcoreMesh(core_axis_name="c", subcore_axis_name="s")
    run = functools.partial(
        pl.kernel,
        mesh=mesh,
        compiler_params=pltpu.CompilerParams(use_tc_tiling_on_sc=False),
        out_type=jax.ShapeDtypeStruct((BATCH * SEQ, EMBED_DIM), jnp.float32),
        scratch_types=[
            pltpu.VMEM((ROWS_PER_WORKER,), jnp.int32),
            pltpu.VMEM((CHUNK, EMBED_DIM), jnp.float32),
            pltpu.VMEM((CHUNK, EMBED_DIM), jnp.float32),
            pltpu.SemaphoreType.DMA,
            pltpu.SemaphoreType.DMA,
            pltpu.SemaphoreType.DMA,
            pltpu.SemaphoreType.DMA,
        ],
    )(_sc_kernel_body)
    out = run(x_flat, token_table, pos_table)
    return out.reshape(BATCH, SEQ, EMBED_DIM)


# speedup vs baseline: 1.5662x; 1.5662x over previous
import functools

import jax
import jax.numpy as jnp
from jax import lax
from jax.experimental import pallas as pl
from jax.experimental.pallas import tpu as pltpu
from jax.experimental.pallas import tpu_sc as plsc

VOCAB = 100000
MAX_LEN = 200
EMBED_DIM = 64
BATCH = 1024
SEQ = 200

NUM_CORES = 2
NUM_SUBCORES = 16
NUM_WORKERS = NUM_CORES * NUM_SUBCORES
ROWS_PER_WORKER = BATCH * SEQ // NUM_WORKERS
CHUNK = 100
CHUNKS_PER_WORKER = ROWS_PER_WORKER // CHUNK
LANES = 16
DGROUPS = EMBED_DIM // LANES
NBUF = 4
LOOKAHEAD = 2


def _sc_kernel_body(x_hbm, tok_hbm, pos_hbm, out_hbm,
                    idx_v, pos_v, bufs, gsems, ssems):
    wid = lax.axis_index("s") * NUM_CORES + lax.axis_index("c")
    base = wid * ROWS_PER_WORKER

    pltpu.sync_copy(x_hbm.at[wid], idx_v)
    pltpu.sync_copy(pos_hbm, pos_v)

    def start_gather(c, slot):
        pltpu.async_copy(tok_hbm.at[idx_v.at[c]], bufs[slot], gsems[slot])

    def wait_gather(slot):
        pltpu.make_async_copy(tok_hbm.at[idx_v.at[0]],
                              bufs[slot], gsems[slot]).wait()

    def start_store(c, slot):
        pltpu.async_copy(bufs[slot], out_hbm.at[pl.ds(base + c * CHUNK, CHUNK)],
                         ssems[slot])

    def wait_store(slot):
        pltpu.make_async_copy(bufs[slot], out_hbm.at[pl.ds(base, CHUNK)],
                              ssems[slot]).wait()

    def add_pos(c, slot):
        buf = bufs[slot]
        s0 = lax.rem(c, SEQ // CHUNK) * CHUNK

        def body(r, carry):
            for k in range(2):
                row = 2 * r + k
                for j in range(DGROUPS):
                    sl = pl.ds(j * LANES, LANES)
                    buf[row, sl] = buf[row, sl] + pos_v[s0 + row, sl]
            return carry

        lax.fori_loop(0, CHUNK // 2, body, 0)

    for c in range(LOOKAHEAD):
        start_gather(c, c)

    def step(g, carry):
        for b in range(NBUF):
            c = g * NBUF + b
            slot = b
            slot_ahead = (b + LOOKAHEAD) % NBUF

            @pl.when(c + LOOKAHEAD < CHUNKS_PER_WORKER)
            def _issue_ahead():
                @pl.when(c >= NBUF - LOOKAHEAD)
                def _drain_old_store():
                    wait_store(slot_ahead)

                start_gather(c + LOOKAHEAD, slot_ahead)

            wait_gather(slot)
            add_pos(c, slot)
            start_store(c, slot)
        return carry

    lax.fori_loop(0, CHUNKS_PER_WORKER // NBUF, step, 0)

    for b in range(NBUF):
        wait_store(b)


@jax.jit
def kernel(x, token_table, pos_table):
    x_flat = x.reshape(NUM_WORKERS, CHUNKS_PER_WORKER, CHUNK).astype(jnp.int32)

    mesh = plsc.VectorSubcoreMesh(core_axis_name="c", subcore_axis_name="s")
    run = functools.partial(
        pl.kernel,
        mesh=mesh,
        compiler_params=pltpu.CompilerParams(use_tc_tiling_on_sc=False),
        out_type=jax.ShapeDtypeStruct((BATCH * SEQ, EMBED_DIM), jnp.float32),
        scratch_types=[
            pltpu.VMEM((CHUNKS_PER_WORKER, CHUNK), jnp.int32),
            pltpu.VMEM((MAX_LEN, EMBED_DIM), jnp.float32),
            tuple(pltpu.VMEM((CHUNK, EMBED_DIM), jnp.float32)
                  for _ in range(NBUF)),
            tuple(pltpu.SemaphoreType.DMA for _ in range(NBUF)),
            tuple(pltpu.SemaphoreType.DMA for _ in range(NBUF)),
        ],
    )(_sc_kernel_body)
    out = run(x_flat, token_table, pos_table)
    return out.reshape(BATCH, SEQ, EMBED_DIM)


# speedup vs baseline: 1.5875x; 1.0135x over previous
import functools

import jax
import jax.numpy as jnp
from jax import lax
from jax.experimental import pallas as pl
from jax.experimental.pallas import tpu as pltpu
from jax.experimental.pallas import tpu_sc as plsc

VOCAB = 100000
MAX_LEN = 200
EMBED_DIM = 64
BATCH = 1024
SEQ = 200

NUM_CORES = 2
NUM_SUBCORES = 16
NUM_WORKERS = NUM_CORES * NUM_SUBCORES
SEQS_PER_WORKER = BATCH // NUM_WORKERS
LANES = 16
DGROUPS = EMBED_DIM // LANES
NBUF = 4
LOOKAHEAD = 2


def _sc_kernel_body(x_hbm, tok_hbm, pos_hbm, out_hbm,
                    idx_v, pos_v, bufs, gsems, ssems):
    wid = lax.axis_index("s") * NUM_CORES + lax.axis_index("c")
    base = wid * SEQS_PER_WORKER

    pltpu.sync_copy(x_hbm.at[pl.ds(base, SEQS_PER_WORKER)], idx_v)
    pltpu.sync_copy(pos_hbm, pos_v)

    def start_gather(q, slot):
        pltpu.async_copy(tok_hbm.at[idx_v.at[q]], bufs[slot], gsems[slot])

    def wait_gather(slot):
        pltpu.make_async_copy(tok_hbm.at[idx_v.at[0]],
                              bufs[slot], gsems[slot]).wait()

    def start_store(q, slot):
        pltpu.async_copy(bufs[slot], out_hbm.at[base + q], ssems[slot])

    def wait_store(slot):
        pltpu.make_async_copy(bufs[slot], out_hbm.at[base], ssems[slot]).wait()

    def add_pos(slot):
        buf = bufs[slot]

        def body(r, carry):
            for k in range(2):
                row = 2 * r + k
                for j in range(DGROUPS):
                    sl = pl.ds(j * LANES, LANES)
                    buf[row, sl] = buf[row, sl] + pos_v[row, sl]
            return carry

        lax.fori_loop(0, SEQ // 2, body, 0)

    for q in range(LOOKAHEAD):
        start_gather(q, q)

    def step(g, carry):
        for b in range(NBUF):
            q = g * NBUF + b
            slot = b
            slot_ahead = (b + LOOKAHEAD) % NBUF

            @pl.when(q + LOOKAHEAD < SEQS_PER_WORKER)
            def _issue_ahead():
                @pl.when(q >= NBUF - LOOKAHEAD)
                def _drain_old_store():
                    wait_store(slot_ahead)

                start_gather(q + LOOKAHEAD, slot_ahead)

            wait_gather(slot)
            add_pos(slot)
            start_store(q, slot)
        return carry

    lax.fori_loop(0, SEQS_PER_WORKER // NBUF, step, 0)

    for b in range(NBUF):
        wait_store(b)


@jax.jit
def kernel(x, token_table, pos_table):
    mesh = plsc.VectorSubcoreMesh(core_axis_name="c", subcore_axis_name="s")
    run = functools.partial(
        pl.kernel,
        mesh=mesh,
        compiler_params=pltpu.CompilerParams(use_tc_tiling_on_sc=False),
        out_type=jax.ShapeDtypeStruct((BATCH, SEQ, EMBED_DIM), jnp.float32),
        scratch_types=[
            pltpu.VMEM((SEQS_PER_WORKER, SEQ), jnp.int32),
            pltpu.VMEM((MAX_LEN, EMBED_DIM), jnp.float32),
            tuple(pltpu.VMEM((SEQ, EMBED_DIM), jnp.float32)
                  for _ in range(NBUF)),
            tuple(pltpu.SemaphoreType.DMA for _ in range(NBUF)),
            tuple(pltpu.SemaphoreType.DMA for _ in range(NBUF)),
        ],
    )(_sc_kernel_body)
    return run(x.astype(jnp.int32), token_table, pos_table)
